# Initial kernel scaffold; baseline (speedup 1.0000x reference)
import jax
import jax.numpy as jnp
from jax.experimental import pallas as pl


def kernel(x, skip_0, skip_1, conv1_w, bn1_g, bn1_b, bn1_m, bn1_v, tconv_w, tconv_b, bn2_g, bn2_b, bn2_m, bn2_v, conv3_w, bn3_g, bn3_b, bn3_m, bn3_v):
    raise NotImplementedError("write your pallas kernel here")


# speedup vs baseline: 1.0302x; 1.0302x over previous
import functools

import jax
import jax.numpy as jnp
from jax.experimental import pallas as pl
from jax.experimental.pallas import tpu as pltpu

_EPS = 1e-5
_TAPS = {0: ((0, 3), (1, 1)), 1: ((1, 2), (2, 0))}
_VMEM_LIMIT = 64 * 1024 * 1024


def _divisor_tile(n, cap):
    for t in range(min(n, cap), 0, -1):
        if n % t == 0:
            return t
    return 1


def _stage1_body(x_ref, sk0_ref, sk1_ref, w_ref, d_ref, b_ref, o_ref):
    wm = w_ref[...]
    u = jnp.dot(wm, x_ref[0], preferred_element_type=jnp.float32)
    u2 = jnp.dot(u, d_ref[...], preferred_element_type=jnp.float32)
    ssum = sk0_ref[0] + sk1_ref[0]
    ys = jnp.dot(wm, ssum, preferred_element_type=jnp.float32)
    o_ref[0] = jnp.maximum(ys + u2 + b_ref[...], 0.0)


def _stage2_body(nsteps, ti, w2, main_ref, top_ref, bot_ref, wt_ref, b2_ref,
                 w3_ref, e0_ref, e1_ref, b3_ref, o_ref):
    _, cm, _ = main_ref.shape
    cout = w3_ref.shape[0]
    j = pl.program_id(1)
    top = jnp.where(j == 0, 0.0, top_ref[0])
    bot = jnp.where(j == nsteps - 1, 0.0, bot_ref[0])
    mid = jnp.concatenate([top, main_ref[0], bot], axis=1)
    lbig = (ti + 2) * w2
    zlane = jnp.zeros((cm, 1), jnp.float32)
    lane = jax.lax.broadcasted_iota(jnp.int32, (1, lbig), 1) % w2
    sh_m = jnp.concatenate([zlane, mid[:, :lbig - 1]], axis=1)
    sh_p = jnp.concatenate([mid[:, 1:], zlane], axis=1)
    xs = [
        jnp.where(lane == 0, 0.0, sh_m),
        mid,
        jnp.where(lane == w2 - 1, 0.0, sh_p),
    ]
    b2 = b2_ref[...]
    w3 = w3_ref[...]
    b3 = b3_ref[...]
    for dy in range(2):
        ys = []
        for dx in range(2):
            patch = jnp.concatenate(
                [xs[rx][:, ry * w2:(ry + ti) * w2]
                 for ry, _ky in _TAPS[dy] for rx, _kx in _TAPS[dx]], axis=0)
            acc = jnp.dot(wt_ref[dy, dx], patch,
                          preferred_element_type=jnp.float32)
            ys.append(jnp.maximum(acc + b2, 0.0))
        yi = (jnp.dot(ys[0].reshape(cm * ti, w2), e0_ref[...],
                      preferred_element_type=jnp.float32)
              + jnp.dot(ys[1].reshape(cm * ti, w2), e1_ref[...],
                        preferred_element_type=jnp.float32))
        z = jnp.dot(w3, yi.reshape(cm, ti * 2 * w2),
                    preferred_element_type=jnp.float32)
        z = jnp.maximum(z + b3, 0.0)
        for iy in range(ti):
            o_ref[0, :, pl.ds(iy * 4 * w2 + dy * 2 * w2, 2 * w2)] = (
                z[:, iy * 2 * w2:(iy + 1) * 2 * w2])


def kernel(x, skip_0, skip_1, conv1_w, bn1_g, bn1_b, bn1_m, bn1_v,
           tconv_w, tconv_b, bn2_g, bn2_b, bn2_m, bn2_v,
           conv3_w, bn3_g, bn3_b, bn3_m, bn3_v):
    N, Cin, H, W = x.shape
    Cm = conv1_w.shape[0]
    Cout = conv3_w.shape[0]
    H2, W2 = 2 * H, 2 * W
    f32 = jnp.float32

    s1 = bn1_g / jnp.sqrt(bn1_v + _EPS)
    b1 = bn1_b - bn1_m * s1
    w1f = conv1_w * s1[:, None]
    s2 = bn2_g / jnp.sqrt(bn2_v + _EPS)
    b2 = bn2_b - bn2_m * s2 + tconv_b * s2
    s3 = bn3_g / jnp.sqrt(bn3_v + _EPS)
    b3 = bn3_b - bn3_m * s3
    w3f = conv3_w * s3[:, None]

    wtt = jnp.transpose(tconv_w, (2, 3, 1, 0))
    wtf = jnp.stack([
        jnp.stack([
            jnp.concatenate([s2[:, None] * wtt[ky, kx]
                             for _ry, ky in _TAPS[dy]
                             for _rx, kx in _TAPS[dx]], axis=1)
            for dx in range(2)])
        for dy in range(2)])

    TH = _divisor_tile(H, 8)
    pidx = jnp.arange(TH * W)
    piy, pj = pidx // W, pidx % W
    d2 = jnp.zeros((TH * W, 2 * TH * W2), f32)
    for rh in range(2):
        for bb in range(2):
            d2 = d2.at[pidx, (2 * piy + rh) * W2 + 2 * pj + bb].set(1.0)
    idx = jnp.arange(W2)
    e0 = jnp.zeros((W2, 2 * W2), f32).at[idx, 2 * idx].set(1.0)
    e1 = jnp.zeros((W2, 2 * W2), f32).at[idx, 2 * idx + 1].set(1.0)

    xf = x.reshape(N, Cin, H * W)
    sk0 = skip_0.reshape(N, Cin, H2 * W2)
    sk1 = skip_1.reshape(N, Cin, H2 * W2)

    h = pl.pallas_call(
        _stage1_body,
        out_shape=jax.ShapeDtypeStruct((N, Cm, H2 * W2), f32),
        grid=(N, H // TH),
        in_specs=[
            pl.BlockSpec((1, Cin, TH * W), lambda n, j: (n, 0, j)),
            pl.BlockSpec((1, Cin, 2 * TH * W2), lambda n, j: (n, 0, j)),
            pl.BlockSpec((1, Cin, 2 * TH * W2), lambda n, j: (n, 0, j)),
            pl.BlockSpec((Cm, Cin), lambda n, j: (0, 0)),
            pl.BlockSpec((TH * W, 2 * TH * W2), lambda n, j: (0, 0)),
            pl.BlockSpec((Cm, 1), lambda n, j: (0, 0)),
        ],
        out_specs=pl.BlockSpec((1, Cm, 2 * TH * W2), lambda n, j: (n, 0, j)),
        compiler_params=pltpu.CompilerParams(
            dimension_semantics=("parallel", "parallel"),
            vmem_limit_bytes=_VMEM_LIMIT),
    )(xf, sk0, sk1, w1f, d2, b1[:, None])

    TI = _divisor_tile(H2, 16)
    nsteps = H2 // TI
    out3 = pl.pallas_call(
        functools.partial(_stage2_body, nsteps, TI, W2),
        out_shape=jax.ShapeDtypeStruct((N, Cout, H2 * 4 * W2), f32),
        grid=(N, nsteps),
        in_specs=[
            pl.BlockSpec((1, Cm, TI * W2), lambda n, j: (n, 0, j)),
            pl.BlockSpec((1, Cm, W2),
                         lambda n, j: (n, 0, jnp.maximum(j * TI - 1, 0))),
            pl.BlockSpec((1, Cm, W2),
                         lambda n, j: (n, 0, jnp.minimum((j + 1) * TI, 2 * H - 1))),
            pl.BlockSpec((2, 2, Cm, 4 * Cm), lambda n, j: (0, 0, 0, 0)),
            pl.BlockSpec((Cm, 1), lambda n, j: (0, 0)),
            pl.BlockSpec((Cout, Cm), lambda n, j: (0, 0)),
            pl.BlockSpec((W2, 2 * W2), lambda n, j: (0, 0)),
            pl.BlockSpec((W2, 2 * W2), lambda n, j: (0, 0)),
            pl.BlockSpec((Cout, 1), lambda n, j: (0, 0)),
        ],
        out_specs=pl.BlockSpec((1, Cout, TI * 4 * W2), lambda n, j: (n, 0, j)),
        compiler_params=pltpu.CompilerParams(
            dimension_semantics=("parallel", "parallel"),
            vmem_limit_bytes=_VMEM_LIMIT),
    )(h, h, h, wtf, b2[:, None], w3f, e0, e1, b3[:, None])
    return out3.reshape(N, Cout, 4 * H, 4 * W)


# speedup vs baseline: 1.0431x; 1.0126x over previous
import functools

import jax
import jax.numpy as jnp
from jax.experimental import pallas as pl
from jax.experimental.pallas import tpu as pltpu

_EPS = 1e-5
_TAPS = {0: ((0, 3), (1, 1)), 1: ((1, 2), (2, 0))}
_VMEM_LIMIT = 64 * 1024 * 1024


def _divisor_tile(n, cap):
    for t in range(min(n, cap), 0, -1):
        if n % t == 0:
            return t
    return 1


def _stage1_body(x_ref, sk0_ref, sk1_ref, w_ref, d_ref, b_ref, o_ref):
    wm = w_ref[...]
    u = jnp.dot(wm, x_ref[0], preferred_element_type=jnp.float32)
    u2 = jnp.dot(u, d_ref[...], preferred_element_type=jnp.float32)
    ssum = sk0_ref[0] + sk1_ref[0]
    ys = jnp.dot(wm, ssum, preferred_element_type=jnp.float32)
    o_ref[0] = jnp.maximum(ys + u2 + b_ref[...], 0.0)


def _stage2_body(nsteps, ti, w2, main_ref, top_ref, bot_ref, wt_ref, b2_ref,
                 w3_ref, e0_ref, e1_ref, b3_ref, o_ref):
    _, cm, _ = main_ref.shape
    cout = w3_ref.shape[0]
    j = pl.program_id(1)
    top = jnp.where(j == 0, 0.0, top_ref[0])
    bot = jnp.where(j == nsteps - 1, 0.0, bot_ref[0])
    mid = jnp.concatenate([top, main_ref[0], bot], axis=1)
    lbig = (ti + 2) * w2
    zlane = jnp.zeros((cm, 1), jnp.float32)
    lane = jax.lax.broadcasted_iota(jnp.int32, (1, lbig), 1) % w2
    sh_m = jnp.concatenate([zlane, mid[:, :lbig - 1]], axis=1)
    sh_p = jnp.concatenate([mid[:, 1:], zlane], axis=1)
    xs = [
        jnp.where(lane == 0, 0.0, sh_m),
        mid,
        jnp.where(lane == w2 - 1, 0.0, sh_p),
    ]
    b2 = b2_ref[...]
    w3 = w3_ref[...]
    b3 = b3_ref[...]
    for dy in range(2):
        ys = []
        for dx in range(2):
            patch = jnp.concatenate(
                [xs[rx][:, ry * w2:(ry + ti) * w2]
                 for ry, _ky in _TAPS[dy] for rx, _kx in _TAPS[dx]], axis=0)
            acc = jnp.dot(wt_ref[dy, dx], patch,
                          preferred_element_type=jnp.float32)
            ys.append(jnp.maximum(acc + b2, 0.0))
        yi = (jnp.dot(ys[0].reshape(cm * ti, w2), e0_ref[...],
                      preferred_element_type=jnp.float32)
              + jnp.dot(ys[1].reshape(cm * ti, w2), e1_ref[...],
                        preferred_element_type=jnp.float32))
        z = jnp.dot(w3, yi.reshape(cm, ti * 2 * w2),
                    preferred_element_type=jnp.float32)
        z = jnp.maximum(z + b3, 0.0)
        for iy in range(ti):
            o_ref[0, :, pl.ds(iy * 4 * w2 + dy * 2 * w2, 2 * w2)] = (
                z[:, iy * 2 * w2:(iy + 1) * 2 * w2])


def kernel(x, skip_0, skip_1, conv1_w, bn1_g, bn1_b, bn1_m, bn1_v,
           tconv_w, tconv_b, bn2_g, bn2_b, bn2_m, bn2_v,
           conv3_w, bn3_g, bn3_b, bn3_m, bn3_v):
    N, Cin, H, W = x.shape
    Cm = conv1_w.shape[0]
    Cout = conv3_w.shape[0]
    H2, W2 = 2 * H, 2 * W
    f32 = jnp.float32

    s1 = bn1_g / jnp.sqrt(bn1_v + _EPS)
    b1 = bn1_b - bn1_m * s1
    w1f = conv1_w * s1[:, None]
    s2 = bn2_g / jnp.sqrt(bn2_v + _EPS)
    b2 = bn2_b - bn2_m * s2 + tconv_b * s2
    s3 = bn3_g / jnp.sqrt(bn3_v + _EPS)
    b3 = bn3_b - bn3_m * s3
    w3f = conv3_w * s3[:, None]

    wtt = jnp.transpose(tconv_w, (2, 3, 1, 0))
    wtf = jnp.stack([
        jnp.stack([
            jnp.concatenate([s2[:, None] * wtt[ky, kx]
                             for _ry, ky in _TAPS[dy]
                             for _rx, kx in _TAPS[dx]], axis=1)
            for dx in range(2)])
        for dy in range(2)])

    TH = _divisor_tile(H, 8)
    pidx = jnp.arange(TH * W)
    piy, pj = pidx // W, pidx % W
    d2 = jnp.zeros((TH * W, 2 * TH * W2), f32)
    for rh in range(2):
        for bb in range(2):
            d2 = d2.at[pidx, (2 * piy + rh) * W2 + 2 * pj + bb].set(1.0)
    idx = jnp.arange(W2)
    e0 = jnp.zeros((W2, 2 * W2), f32).at[idx, 2 * idx].set(1.0)
    e1 = jnp.zeros((W2, 2 * W2), f32).at[idx, 2 * idx + 1].set(1.0)

    xf = x.reshape(N, Cin, H * W)
    sk0 = skip_0.reshape(N, Cin, H2 * W2)
    sk1 = skip_1.reshape(N, Cin, H2 * W2)

    h = pl.pallas_call(
        _stage1_body,
        out_shape=jax.ShapeDtypeStruct((N, Cm, H2 * W2), f32),
        grid=(N, H // TH),
        in_specs=[
            pl.BlockSpec((1, Cin, TH * W), lambda n, j: (n, 0, j)),
            pl.BlockSpec((1, Cin, 2 * TH * W2), lambda n, j: (n, 0, j)),
            pl.BlockSpec((1, Cin, 2 * TH * W2), lambda n, j: (n, 0, j)),
            pl.BlockSpec((Cm, Cin), lambda n, j: (0, 0)),
            pl.BlockSpec((TH * W, 2 * TH * W2), lambda n, j: (0, 0)),
            pl.BlockSpec((Cm, 1), lambda n, j: (0, 0)),
        ],
        out_specs=pl.BlockSpec((1, Cm, 2 * TH * W2), lambda n, j: (n, 0, j)),
        compiler_params=pltpu.CompilerParams(
            dimension_semantics=("parallel", "parallel"),
            vmem_limit_bytes=_VMEM_LIMIT),
    )(xf, sk0, sk1, w1f, d2, b1[:, None])

    TI = _divisor_tile(H2, 32)
    nsteps = H2 // TI
    out3 = pl.pallas_call(
        functools.partial(_stage2_body, nsteps, TI, W2),
        out_shape=jax.ShapeDtypeStruct((N, Cout, H2 * 4 * W2), f32),
        grid=(N, nsteps),
        in_specs=[
            pl.BlockSpec((1, Cm, TI * W2), lambda n, j: (n, 0, j)),
            pl.BlockSpec((1, Cm, W2),
                         lambda n, j: (n, 0, jnp.maximum(j * TI - 1, 0))),
            pl.BlockSpec((1, Cm, W2),
                         lambda n, j: (n, 0, jnp.minimum((j + 1) * TI, 2 * H - 1))),
            pl.BlockSpec((2, 2, Cm, 4 * Cm), lambda n, j: (0, 0, 0, 0)),
            pl.BlockSpec((Cm, 1), lambda n, j: (0, 0)),
            pl.BlockSpec((Cout, Cm), lambda n, j: (0, 0)),
            pl.BlockSpec((W2, 2 * W2), lambda n, j: (0, 0)),
            pl.BlockSpec((W2, 2 * W2), lambda n, j: (0, 0)),
            pl.BlockSpec((Cout, 1), lambda n, j: (0, 0)),
        ],
        out_specs=pl.BlockSpec((1, Cout, TI * 4 * W2), lambda n, j: (n, 0, j)),
        compiler_params=pltpu.CompilerParams(
            dimension_semantics=("parallel", "parallel"),
            vmem_limit_bytes=_VMEM_LIMIT),
    )(h, h, h, wtf, b2[:, None], w3f, e0, e1, b3[:, None])
    return out3.reshape(N, Cout, 4 * H, 4 * W)
